# baseline (speedup 1.0000x reference)
import functools

import jax
import jax.numpy as jnp
from jax import lax
from jax.experimental import pallas as pl
from jax.experimental.pallas import tpu as pltpu
from jax.experimental.pallas import tpu_sc as plsc

N = 10000
E = 320000
D = 128
DE = 16
DH = 128
DEH = 16

NC = 2
NS = 16
LANES = 16

N_PAD = 10240
BLK = 128
BLKD = 64
NBLK_ALL = E // BLK
NBLKD_ALL = E // BLKD
ROWS_PT = N_PAD // NS


def _tc_node_pre_body(x_ref, wnp_ref, bnp_ref, we0_ref, we2_ref,
                      h_ref, hs_ref, hd_ref):
    hb = jnp.dot(x_ref[...], wnp_ref[...],
                 preferred_element_type=jnp.float32) + bnp_ref[...]
    h_ref[...] = hb
    hs_ref[...] = jnp.dot(hb, we0_ref[...], preferred_element_type=jnp.float32)
    hd_ref[...] = jnp.dot(hb, we2_ref[...], preferred_element_type=jnp.float32)


def _tc_node_pre(x, W_np, b_np, We0, We2):
    blk = 1000
    return pl.pallas_call(
        _tc_node_pre_body,
        grid=(N // blk,),
        in_specs=[
            pl.BlockSpec((blk, D), lambda i: (i, 0)),
            pl.BlockSpec((D, DH), lambda i: (0, 0)),
            pl.BlockSpec((DH,), lambda i: (0,)),
            pl.BlockSpec((DH, DE), lambda i: (0, 0)),
            pl.BlockSpec((DH, DE), lambda i: (0, 0)),
        ],
        out_specs=[
            pl.BlockSpec((blk, DH), lambda i: (i, 0)),
            pl.BlockSpec((blk, DE), lambda i: (i, 0)),
            pl.BlockSpec((blk, DE), lambda i: (i, 0)),
        ],
        out_shape=[
            jax.ShapeDtypeStruct((N, DH), jnp.float32),
            jax.ShapeDtypeStruct((N, DE), jnp.float32),
            jax.ShapeDtypeStruct((N, DE), jnp.float32),
        ],
    )(x, W_np, b_np, We0, We2)


def _tc_edge_pre_body(ea_ref, wep_ref, bep_ref, we1_ref, bet_ref,
                      e_ref, ee_ref):
    eb = jnp.dot(ea_ref[...], wep_ref[...],
                 preferred_element_type=jnp.float32) + bep_ref[...]
    e_ref[...] = eb
    ee_ref[...] = jnp.dot(eb, we1_ref[...],
                          preferred_element_type=jnp.float32) + bet_ref[...]


def _tc_edge_pre(edge_attr, W_ep, b_ep, We1, b_et):
    eblk = 8000
    return pl.pallas_call(
        _tc_edge_pre_body,
        grid=(E // eblk,),
        in_specs=[
            pl.BlockSpec((eblk, DE), lambda i: (i, 0)),
            pl.BlockSpec((DE, DEH), lambda i: (0, 0)),
            pl.BlockSpec((DEH,), lambda i: (0,)),
            pl.BlockSpec((DEH, DE), lambda i: (0, 0)),
            pl.BlockSpec((DE,), lambda i: (0,)),
        ],
        out_specs=[
            pl.BlockSpec((eblk, DEH), lambda i: (i, 0)),
            pl.BlockSpec((eblk, DE), lambda i: (i, 0)),
        ],
        out_shape=[
            jax.ShapeDtypeStruct((E, DEH), jnp.float32),
            jax.ShapeDtypeStruct((E, DE), jnp.float32),
        ],
    )(edge_attr, W_ep, b_ep, We1, b_et)


_ZERO16 = functools.partial(jnp.zeros, (LANES,), jnp.float32)


def _sc_graph_body(h_hbm, e_hbm, src_hbm, dst_hbm, hs_hbm, hd_hbm, ee_hbm,
                   shi_hbm, sho_hbm, sei_hbm, seo_hbm, eo_hbm, deg_sh,
                   acc_h, acc_e,
                   rows0_v, rows1_v, erow0_v, erow1_v, gi0_v, gi1_v,
                   si0_v, si1_v, gie_v, sie_v, hist_v, degr_v, tmp_v,
                   a_v, b_v, c_v,
                   gsem0, gsem1, shsem0, shsem1, sesem0, sesem1, sem, sem2):
    c = lax.axis_index("c")
    s = lax.axis_index("s")
    rows = (rows0_v, rows1_v)
    erow = (erow0_v, erow1_v)
    gi = (gi0_v, gi1_v)
    si = (si0_v, si1_v)
    gsem = (gsem0, gsem1)
    shsem = (shsem0, shsem1)
    sesem = (sesem0, sesem1)

    def zero_rows(i, _):
        for k in range(DH // LANES):
            rows0_v[i, pl.ds(k * LANES, LANES)] = _ZERO16()
        erow0_v[i] = _ZERO16()
        return ()

    lax.fori_loop(0, BLKD, zero_rows, ())

    def zero_acc(g, _):
        r = s * ROWS_PT + g * BLKD
        pltpu.sync_copy(rows0_v, acc_h.at[pl.ds(r, BLKD)])
        pltpu.sync_copy(erow0_v, acc_e.at[pl.ds(r, BLKD)])
        return ()

    lax.fori_loop(0, ROWS_PT // BLKD, zero_acc, ())

    def zero_hist(i, _):
        hist_v[pl.ds(i * LANES, LANES)] = _ZERO16()
        return ()

    lax.fori_loop(0, N_PAD // LANES, zero_hist, ())
    plsc.subcore_barrier()

    my_nblk = jnp.where(s < NBLKD_ALL - (NBLKD_ALL // NS) * NS,
                        NBLKD_ALL // NS + 1, NBLKD_ALL // NS)
    nb2 = (NBLKD_ALL // NS + 2) // 2

    def run_direction(gref, sref):
        def pair_body(i, _):
            for p in (0, 1):
                b = i * 2 + p

                @pl.when(b < my_nblk)
                def _():
                    @pl.when(b >= 2)
                    def _():
                        pltpu.make_async_copy(
                            rows[p], acc_h.at[si[p].at[0]], shsem[p]).wait()
                        pltpu.make_async_copy(
                            erow[p], acc_e.at[si[p].at[0]], sesem[p]).wait()

                    off = (b * NS + s) * BLKD
                    pltpu.sync_copy(gref.at[pl.ds(off, BLKD)], gi[p])
                    pltpu.sync_copy(sref.at[pl.ds(off, BLKD)], si[p].at[0])
                    cp = pltpu.async_copy(h_hbm.at[gi[p]], rows[p], gsem[p])
                    pltpu.sync_copy(e_hbm.at[pl.ds(off, BLKD)], erow[p])
                    for j in range(BLKD // LANES):
                        iv = si[p][0, pl.ds(j * LANES, LANES)]
                        plsc.addupdate_scatter(hist_v, [iv],
                                               jnp.ones((LANES,), jnp.float32))
                    cp.wait()
                    pltpu.async_copy(rows[p], acc_h.at[si[p].at[0]],
                                     shsem[p], add=True)
                    pltpu.async_copy(erow[p], acc_e.at[si[p].at[0]],
                                     sesem[p], add=True)

            return ()

        lax.fori_loop(0, nb2, pair_body, ())
        for p in (0, 1):
            pltpu.make_async_copy(
                rows[p], acc_h.at[si[p].at[0]], shsem[p]).wait()
            pltpu.make_async_copy(
                erow[p], acc_e.at[si[p].at[0]], sesem[p]).wait()

    @pl.when(c == 0)
    def _():
        run_direction(src_hbm, dst_hbm)

    @pl.when(c == 1)
    def _():
        run_direction(dst_hbm, src_hbm)

    pltpu.sync_copy(hist_v, deg_sh.at[c, s])
    plsc.subcore_barrier()

    r0 = s * ROWS_PT

    def red_init(i, _):
        degr_v[pl.ds(i * LANES, LANES)] = _ZERO16()
        return ()

    lax.fori_loop(0, ROWS_PT // LANES, red_init, ())

    def red_j(j, _):
        pltpu.sync_copy(deg_sh.at[c, j, pl.ds(r0, ROWS_PT)], tmp_v)

        def addk(k, _):
            sl = pl.ds(k * LANES, LANES)
            degr_v[sl] = degr_v[sl] + tmp_v[sl]
            return ()

        lax.fori_loop(0, ROWS_PT // LANES, addk, ())
        return ()

    lax.fori_loop(0, NS, red_j, ())

    def scale_big(g5, _):
        rr = r0 + g5 * BLKD
        pltpu.sync_copy(acc_h.at[pl.ds(rr, BLKD)], rows0_v)
        pltpu.sync_copy(acc_e.at[pl.ds(rr, BLKD)], erow0_v)

        def scale_grp(gg, _):
            deg16 = degr_v[pl.ds(g5 * BLKD + gg * LANES, LANES)]
            r16 = 1.0 / jnp.maximum(deg16, 1.0)
            for i in range(LANES):
                r_s = r16[i]
                row = gg * LANES + i
                for k in range(DH // LANES):
                    sl = pl.ds(k * LANES, LANES)
                    rows0_v[row, sl] = rows0_v[row, sl] * r_s
                erow0_v[row] = erow0_v[row] * r_s
            return ()

        lax.fori_loop(0, BLKD // LANES, scale_grp, ())

        @pl.when(c == 0)
        def _():
            pltpu.sync_copy(rows0_v, shi_hbm.at[pl.ds(rr, BLKD)])
            pltpu.sync_copy(erow0_v, sei_hbm.at[pl.ds(rr, BLKD)])

        @pl.when(c == 1)
        def _():
            pltpu.sync_copy(rows0_v, sho_hbm.at[pl.ds(rr, BLKD)])
            pltpu.sync_copy(erow0_v, seo_hbm.at[pl.ds(rr, BLKD)])

        return ()

    lax.fori_loop(0, ROWS_PT // BLKD, scale_big, ())

    wid = s * NC + c
    NW = NC * NS
    my_eo_nblk = jnp.where(wid < NBLK_ALL - (NBLK_ALL // NW) * NW,
                           NBLK_ALL // NW + 1, NBLK_ALL // NW)

    def eo_body(b, _):
        off = (b * NW + wid) * BLK
        pltpu.sync_copy(src_hbm.at[pl.ds(off, BLK)], gie_v)
        pltpu.sync_copy(dst_hbm.at[pl.ds(off, BLK)], sie_v.at[0])
        cp1 = pltpu.async_copy(hs_hbm.at[gie_v], a_v, sem)
        cp2 = pltpu.async_copy(hd_hbm.at[sie_v.at[0]], b_v, sem2)
        pltpu.sync_copy(ee_hbm.at[pl.ds(off, BLK)], c_v)
        cp1.wait()
        cp2.wait()

        def rowadd(i, _):
            a_v[i] = a_v[i] + b_v[i] + c_v[i]
            return ()

        lax.fori_loop(0, BLK, rowadd, ())
        pltpu.sync_copy(a_v, eo_hbm.at[pl.ds(off, BLK)])
        return ()

    lax.fori_loop(0, my_eo_nblk, eo_body, ())


_sc_graph = functools.partial(
    pl.kernel,
    out_type=[
        jax.ShapeDtypeStruct((N_PAD, DH), jnp.float32),
        jax.ShapeDtypeStruct((N_PAD, DH), jnp.float32),
        jax.ShapeDtypeStruct((N_PAD, DEH), jnp.float32),
        jax.ShapeDtypeStruct((N_PAD, DEH), jnp.float32),
        jax.ShapeDtypeStruct((E, DE), jnp.float32),
        jax.ShapeDtypeStruct((NC, NS, N_PAD), jnp.float32),
    ],
    mesh=plsc.VectorSubcoreMesh(core_axis_name="c", subcore_axis_name="s"),
    compiler_params=pltpu.CompilerParams(
        needs_layout_passes=False, use_tc_tiling_on_sc=False),
    scratch_types=[
        pltpu.VMEM_SHARED((N_PAD, DH), jnp.float32),
        pltpu.VMEM_SHARED((N_PAD, DEH), jnp.float32),
        pltpu.VMEM((BLKD, DH), jnp.float32),
        pltpu.VMEM((BLKD, DH), jnp.float32),
        pltpu.VMEM((BLKD, DEH), jnp.float32),
        pltpu.VMEM((BLKD, DEH), jnp.float32),
        pltpu.VMEM((BLKD,), jnp.int32),
        pltpu.VMEM((BLKD,), jnp.int32),
        pltpu.VMEM((1, BLKD), jnp.int32),
        pltpu.VMEM((1, BLKD), jnp.int32),
        pltpu.VMEM((BLK,), jnp.int32),
        pltpu.VMEM((1, BLK), jnp.int32),
        pltpu.VMEM((N_PAD,), jnp.float32),
        pltpu.VMEM((ROWS_PT,), jnp.float32),
        pltpu.VMEM((ROWS_PT,), jnp.float32),
        pltpu.VMEM((BLK, DE), jnp.float32),
        pltpu.VMEM((BLK, DE), jnp.float32),
        pltpu.VMEM((BLK, DE), jnp.float32),
        pltpu.SemaphoreType.DMA,
        pltpu.SemaphoreType.DMA,
        pltpu.SemaphoreType.DMA,
        pltpu.SemaphoreType.DMA,
        pltpu.SemaphoreType.DMA,
        pltpu.SemaphoreType.DMA,
        pltpu.SemaphoreType.DMA,
        pltpu.SemaphoreType.DMA,
    ],
)(_sc_graph_body)


def _tc_node_post_body(shi_ref, sei_ref, h_ref, sho_ref, seo_ref,
                       w1_ref, w2_ref, w3_ref, w4_ref, w5_ref, bnt_ref,
                       out_ref):
    acc = jnp.dot(shi_ref[...], w1_ref[...], preferred_element_type=jnp.float32)
    acc = acc + jnp.dot(sei_ref[...], w2_ref[...],
                        preferred_element_type=jnp.float32)
    acc = acc + jnp.dot(h_ref[...], w3_ref[...],
                        preferred_element_type=jnp.float32)
    acc = acc + jnp.dot(sho_ref[...], w4_ref[...],
                        preferred_element_type=jnp.float32)
    acc = acc + jnp.dot(seo_ref[...], w5_ref[...],
                        preferred_element_type=jnp.float32)
    out_ref[...] = acc + bnt_ref[...]


def _tc_node_post(shi, sei, h, sho, seo, W1, W2, W3, W4, W5, b_nt):
    blk = 1000
    return pl.pallas_call(
        _tc_node_post_body,
        grid=(N // blk,),
        in_specs=[
            pl.BlockSpec((blk, DH), lambda i: (i, 0)),
            pl.BlockSpec((blk, DEH), lambda i: (i, 0)),
            pl.BlockSpec((blk, DH), lambda i: (i, 0)),
            pl.BlockSpec((blk, DH), lambda i: (i, 0)),
            pl.BlockSpec((blk, DEH), lambda i: (i, 0)),
            pl.BlockSpec((DH, D), lambda i: (0, 0)),
            pl.BlockSpec((DEH, D), lambda i: (0, 0)),
            pl.BlockSpec((DH, D), lambda i: (0, 0)),
            pl.BlockSpec((DH, D), lambda i: (0, 0)),
            pl.BlockSpec((DEH, D), lambda i: (0, 0)),
            pl.BlockSpec((D,), lambda i: (0,)),
        ],
        out_specs=pl.BlockSpec((blk, D), lambda i: (i, 0)),
        out_shape=jax.ShapeDtypeStruct((N, D), jnp.float32),
    )(shi, sei, h, sho, seo, W1, W2, W3, W4, W5, b_nt)


def kernel(x, edge_index, edge_attr, W_np, b_np, W_ep, b_ep,
           W_nt, b_nt, W_et, b_et):
    src = edge_index[0]
    dst = edge_index[1]

    h, hs_et, hd_et = _tc_node_pre(
        x, W_np, b_np, W_et[0:DH], W_et[DH + DEH:])
    e, ee = _tc_edge_pre(edge_attr, W_ep, b_ep, W_et[DH:DH + DEH], b_et)

    shi, sho, sei, seo, eo, _ = _sc_graph(
        h, e, src, dst, hs_et, hd_et, ee)

    node_out = _tc_node_post(
        shi, sei, h, sho, seo,
        W_nt[0:DH], W_nt[DH:DH + DEH], W_nt[DH + DEH:2 * DH + DEH],
        W_nt[2 * DH + DEH:3 * DH + DEH], W_nt[3 * DH + DEH:], b_nt)

    return node_out, eo


# speedup vs baseline: 1.1037x; 1.1037x over previous
import functools

import jax
import jax.numpy as jnp
from jax import lax
from jax.experimental import pallas as pl
from jax.experimental.pallas import tpu as pltpu
from jax.experimental.pallas import tpu_sc as plsc

N = 10000
E = 320000
D = 128
DE = 16
DH = 128
DEH = 16

NC = 2
NS = 16
LANES = 16

N_PAD = 10240
BLK = 128
BLKD = 64
NBLK_ALL = E // BLK
NBLKD_ALL = E // BLKD
ROWS_PT = N_PAD // NS


def _tc_node_pre_body(x_ref, wnp_ref, bnp_ref, we0_ref, we2_ref,
                      h_ref, hs_ref, hd_ref):
    hb = jnp.dot(x_ref[...], wnp_ref[...],
                 preferred_element_type=jnp.float32) + bnp_ref[...]
    h_ref[...] = hb
    hs_ref[...] = jnp.dot(hb, we0_ref[...], preferred_element_type=jnp.float32)
    hd_ref[...] = jnp.dot(hb, we2_ref[...], preferred_element_type=jnp.float32)


def _tc_node_pre(x, W_np, b_np, We0, We2):
    blk = 1000
    return pl.pallas_call(
        _tc_node_pre_body,
        grid=(N // blk,),
        in_specs=[
            pl.BlockSpec((blk, D), lambda i: (i, 0)),
            pl.BlockSpec((D, DH), lambda i: (0, 0)),
            pl.BlockSpec((DH,), lambda i: (0,)),
            pl.BlockSpec((DH, DE), lambda i: (0, 0)),
            pl.BlockSpec((DH, DE), lambda i: (0, 0)),
        ],
        out_specs=[
            pl.BlockSpec((blk, DH), lambda i: (i, 0)),
            pl.BlockSpec((blk, DE), lambda i: (i, 0)),
            pl.BlockSpec((blk, DE), lambda i: (i, 0)),
        ],
        out_shape=[
            jax.ShapeDtypeStruct((N, DH), jnp.float32),
            jax.ShapeDtypeStruct((N, DE), jnp.float32),
            jax.ShapeDtypeStruct((N, DE), jnp.float32),
        ],
    )(x, W_np, b_np, We0, We2)


def _tc_edge_pre_body(ea_ref, wep_ref, bep_ref, we1_ref, bet_ref,
                      e_ref, ee_ref):
    eb = jnp.dot(ea_ref[...], wep_ref[...],
                 preferred_element_type=jnp.float32) + bep_ref[...]
    e_ref[...] = eb
    ee_ref[...] = jnp.dot(eb, we1_ref[...],
                          preferred_element_type=jnp.float32) + bet_ref[...]


def _tc_edge_pre(edge_attr, W_ep, b_ep, We1, b_et):
    eblk = 8000
    return pl.pallas_call(
        _tc_edge_pre_body,
        grid=(E // eblk,),
        in_specs=[
            pl.BlockSpec((eblk, DE), lambda i: (i, 0)),
            pl.BlockSpec((DE, DEH), lambda i: (0, 0)),
            pl.BlockSpec((DEH,), lambda i: (0,)),
            pl.BlockSpec((DEH, DE), lambda i: (0, 0)),
            pl.BlockSpec((DE,), lambda i: (0,)),
        ],
        out_specs=[
            pl.BlockSpec((eblk, DEH), lambda i: (i, 0)),
            pl.BlockSpec((eblk, DE), lambda i: (i, 0)),
        ],
        out_shape=[
            jax.ShapeDtypeStruct((E, DEH), jnp.float32),
            jax.ShapeDtypeStruct((E, DE), jnp.float32),
        ],
    )(edge_attr, W_ep, b_ep, We1, b_et)


_ZERO16 = functools.partial(jnp.zeros, (LANES,), jnp.float32)


def _sc_graph_body(h_hbm, e_hbm, src_hbm, dst_hbm, hs_hbm, hd_hbm, ee_hbm,
                   shi_hbm, sho_hbm, sei_hbm, seo_hbm, eo_hbm, deg_sh,
                   acc_h, acc_e,
                   rows_v, erow_v, epack_v, opack_v, gi_v, si_v,
                   hist_v, degr_v, tmp_v, a_v, b_v, sem, sem2):
    c = lax.axis_index("c")
    s = lax.axis_index("s")

    def zero_rows(i, _):
        for k in range(DH // LANES):
            rows_v[i, pl.ds(k * LANES, LANES)] = _ZERO16()
        erow_v[i] = _ZERO16()
        return ()

    lax.fori_loop(0, BLK, zero_rows, ())

    def zero_acc(g, _):
        r = s * ROWS_PT + g * BLK
        pltpu.sync_copy(rows_v, acc_h.at[pl.ds(r, BLK)])
        pltpu.sync_copy(erow_v, acc_e.at[pl.ds(r, BLK)])
        return ()

    lax.fori_loop(0, ROWS_PT // BLK, zero_acc, ())

    def zero_hist(i, _):
        hist_v[pl.ds(i * LANES, LANES)] = _ZERO16()
        return ()

    lax.fori_loop(0, N_PAD // LANES, zero_hist, ())
    plsc.subcore_barrier()

    my_nblk = jnp.where(s < NBLK_ALL - (NBLK_ALL // NS) * NS,
                        NBLK_ALL // NS + 1, NBLK_ALL // NS)

    def run_direction(gref, sref):
        def blk_body(b, _):
            off = (b * NS + s) * BLK
            pltpu.sync_copy(gref.at[pl.ds(off, BLK)], gi_v)
            pltpu.sync_copy(sref.at[pl.ds(off, BLK)], si_v.at[0])
            cp = pltpu.async_copy(h_hbm.at[gi_v], rows_v, sem)
            pltpu.sync_copy(e_hbm.at[pl.ds(off // 8, BLK // 8)], epack_v)

            def unpack_i(i, _):
                for j in range(8):
                    erow_v[i * 8 + j] = epack_v[i, pl.ds(j * LANES, LANES)]
                return ()

            lax.fori_loop(0, BLK // 8, unpack_i, ())

            for j in range(BLK // LANES):
                iv = si_v[0, pl.ds(j * LANES, LANES)]
                plsc.addupdate_scatter(hist_v, [iv],
                                       jnp.ones((LANES,), jnp.float32))
            cp.wait()
            pltpu.sync_copy(rows_v, acc_h.at[si_v.at[0]], add=True)
            pltpu.sync_copy(erow_v, acc_e.at[si_v.at[0]], add=True)
            return ()

        lax.fori_loop(0, my_nblk, blk_body, ())

    @pl.when(c == 0)
    def _():
        run_direction(src_hbm, dst_hbm)

    @pl.when(c == 1)
    def _():
        run_direction(dst_hbm, src_hbm)

    pltpu.sync_copy(hist_v, deg_sh.at[c, s])
    plsc.subcore_barrier()

    r0 = s * ROWS_PT

    def red_init(i, _):
        degr_v[pl.ds(i * LANES, LANES)] = _ZERO16()
        return ()

    lax.fori_loop(0, ROWS_PT // LANES, red_init, ())

    def red_j(j, _):
        pltpu.sync_copy(deg_sh.at[c, j, pl.ds(r0, ROWS_PT)], tmp_v)

        def addk(k, _):
            sl = pl.ds(k * LANES, LANES)
            degr_v[sl] = degr_v[sl] + tmp_v[sl]
            return ()

        lax.fori_loop(0, ROWS_PT // LANES, addk, ())
        return ()

    lax.fori_loop(0, NS, red_j, ())

    def scale_big(g5, _):
        rr = r0 + g5 * BLK
        pltpu.sync_copy(acc_h.at[pl.ds(rr, BLK)], rows_v)
        pltpu.sync_copy(acc_e.at[pl.ds(rr, BLK)], erow_v)

        def scale_grp(gg, _):
            deg16 = degr_v[pl.ds(g5 * BLK + gg * LANES, LANES)]
            r16 = 1.0 / jnp.maximum(deg16, 1.0)
            for i in range(LANES):
                r_s = r16[i]
                row = gg * LANES + i
                for k in range(DH // LANES):
                    sl = pl.ds(k * LANES, LANES)
                    rows_v[row, sl] = rows_v[row, sl] * r_s
                erow_v[row] = erow_v[row] * r_s
            return ()

        lax.fori_loop(0, BLK // LANES, scale_grp, ())

        @pl.when(c == 0)
        def _():
            pltpu.sync_copy(rows_v, shi_hbm.at[pl.ds(rr, BLK)])
            pltpu.sync_copy(erow_v, sei_hbm.at[pl.ds(rr, BLK)])

        @pl.when(c == 1)
        def _():
            pltpu.sync_copy(rows_v, sho_hbm.at[pl.ds(rr, BLK)])
            pltpu.sync_copy(erow_v, seo_hbm.at[pl.ds(rr, BLK)])

        return ()

    lax.fori_loop(0, ROWS_PT // BLK, scale_big, ())

    wid = s * NC + c
    NW = NC * NS
    my_eo_nblk = jnp.where(wid < NBLK_ALL - (NBLK_ALL // NW) * NW,
                           NBLK_ALL // NW + 1, NBLK_ALL // NW)

    def eo_body(b, _):
        off = (b * NW + wid) * BLK
        pltpu.sync_copy(src_hbm.at[pl.ds(off, BLK)], gi_v)
        pltpu.sync_copy(dst_hbm.at[pl.ds(off, BLK)], si_v.at[0])
        cp1 = pltpu.async_copy(hs_hbm.at[gi_v], a_v, sem)
        cp2 = pltpu.async_copy(hd_hbm.at[si_v.at[0]], b_v, sem2)
        pltpu.sync_copy(ee_hbm.at[pl.ds(off // 8, BLK // 8)], epack_v)
        cp1.wait()
        cp2.wait()

        def packadd_i(i, _):
            for j in range(8):
                sl = pl.ds(j * LANES, LANES)
                opack_v[i, sl] = epack_v[i, sl] + a_v[i * 8 + j] + b_v[i * 8 + j]
            return ()

        lax.fori_loop(0, BLK // 8, packadd_i, ())
        pltpu.sync_copy(opack_v, eo_hbm.at[pl.ds(off // 8, BLK // 8)])
        return ()

    lax.fori_loop(0, my_eo_nblk, eo_body, ())


_sc_graph = functools.partial(
    pl.kernel,
    out_type=[
        jax.ShapeDtypeStruct((N_PAD, DH), jnp.float32),
        jax.ShapeDtypeStruct((N_PAD, DH), jnp.float32),
        jax.ShapeDtypeStruct((N_PAD, DEH), jnp.float32),
        jax.ShapeDtypeStruct((N_PAD, DEH), jnp.float32),
        jax.ShapeDtypeStruct((E // 8, D), jnp.float32),
        jax.ShapeDtypeStruct((NC, NS, N_PAD), jnp.float32),
    ],
    mesh=plsc.VectorSubcoreMesh(core_axis_name="c", subcore_axis_name="s"),
    compiler_params=pltpu.CompilerParams(
        needs_layout_passes=False, use_tc_tiling_on_sc=False),
    scratch_types=[
        pltpu.VMEM_SHARED((N_PAD, DH), jnp.float32),
        pltpu.VMEM_SHARED((N_PAD, DEH), jnp.float32),
        pltpu.VMEM((BLK, DH), jnp.float32),
        pltpu.VMEM((BLK, DEH), jnp.float32),
        pltpu.VMEM((BLK // 8, D), jnp.float32),
        pltpu.VMEM((BLK // 8, D), jnp.float32),
        pltpu.VMEM((BLK,), jnp.int32),
        pltpu.VMEM((1, BLK), jnp.int32),
        pltpu.VMEM((N_PAD,), jnp.float32),
        pltpu.VMEM((ROWS_PT,), jnp.float32),
        pltpu.VMEM((ROWS_PT,), jnp.float32),
        pltpu.VMEM((BLK, DE), jnp.float32),
        pltpu.VMEM((BLK, DE), jnp.float32),
        pltpu.SemaphoreType.DMA,
        pltpu.SemaphoreType.DMA,
    ],
)(_sc_graph_body)


def _tc_node_post_body(shi_ref, sei_ref, h_ref, sho_ref, seo_ref,
                       w1_ref, w2_ref, w3_ref, w4_ref, w5_ref, bnt_ref,
                       out_ref):
    acc = jnp.dot(shi_ref[...], w1_ref[...], preferred_element_type=jnp.float32)
    acc = acc + jnp.dot(sei_ref[...], w2_ref[...],
                        preferred_element_type=jnp.float32)
    acc = acc + jnp.dot(h_ref[...], w3_ref[...],
                        preferred_element_type=jnp.float32)
    acc = acc + jnp.dot(sho_ref[...], w4_ref[...],
                        preferred_element_type=jnp.float32)
    acc = acc + jnp.dot(seo_ref[...], w5_ref[...],
                        preferred_element_type=jnp.float32)
    out_ref[...] = acc + bnt_ref[...]


def _tc_node_post(shi, sei, h, sho, seo, W1, W2, W3, W4, W5, b_nt):
    blk = 1000
    return pl.pallas_call(
        _tc_node_post_body,
        grid=(N // blk,),
        in_specs=[
            pl.BlockSpec((blk, DH), lambda i: (i, 0)),
            pl.BlockSpec((blk, DEH), lambda i: (i, 0)),
            pl.BlockSpec((blk, DH), lambda i: (i, 0)),
            pl.BlockSpec((blk, DH), lambda i: (i, 0)),
            pl.BlockSpec((blk, DEH), lambda i: (i, 0)),
            pl.BlockSpec((DH, D), lambda i: (0, 0)),
            pl.BlockSpec((DEH, D), lambda i: (0, 0)),
            pl.BlockSpec((DH, D), lambda i: (0, 0)),
            pl.BlockSpec((DH, D), lambda i: (0, 0)),
            pl.BlockSpec((DEH, D), lambda i: (0, 0)),
            pl.BlockSpec((D,), lambda i: (0,)),
        ],
        out_specs=pl.BlockSpec((blk, D), lambda i: (i, 0)),
        out_shape=jax.ShapeDtypeStruct((N, D), jnp.float32),
    )(shi, sei, h, sho, seo, W1, W2, W3, W4, W5, b_nt)


def kernel(x, edge_index, edge_attr, W_np, b_np, W_ep, b_ep,
           W_nt, b_nt, W_et, b_et):
    src = edge_index[0]
    dst = edge_index[1]

    h, hs_et, hd_et = _tc_node_pre(
        x, W_np, b_np, W_et[0:DH], W_et[DH + DEH:])
    e, ee = _tc_edge_pre(edge_attr, W_ep, b_ep, W_et[DH:DH + DEH], b_et)
    e128 = e.reshape(E // 8, D)
    ee128 = ee.reshape(E // 8, D)

    shi, sho, sei, seo, eo128, _ = _sc_graph(
        h, e128, src, dst, hs_et, hd_et, ee128)

    edge_out = eo128.reshape(E, DE)
    node_out = _tc_node_post(
        shi, sei, h, sho, seo,
        W_nt[0:DH], W_nt[DH:DH + DEH], W_nt[DH + DEH:2 * DH + DEH],
        W_nt[2 * DH + DEH:3 * DH + DEH], W_nt[3 * DH + DEH:], b_nt)

    return node_out, edge_out


# speedup vs baseline: 1.7103x; 1.5496x over previous
import functools

import jax
import jax.numpy as jnp
from jax import lax
from jax.experimental import pallas as pl
from jax.experimental.pallas import tpu as pltpu
from jax.experimental.pallas import tpu_sc as plsc

N = 10000
E = 320000
D = 128
DE = 16
DH = 128
DEH = 16

NC = 2
NS = 16
LANES = 16

N_PAD = 10240
BLK = 128
BLKD = 64
NBLK_ALL = E // BLK
NBLKD_ALL = E // BLKD
ROWS_PT = N_PAD // NS


def _tc_node_pre_body(x_ref, wnp_ref, bnp_ref, we0_ref, we2_ref,
                      h_ref, hs_ref, hd_ref):
    hb = jnp.dot(x_ref[...], wnp_ref[...],
                 preferred_element_type=jnp.float32) + bnp_ref[...]
    h_ref[...] = hb
    hs_ref[...] = jnp.dot(hb, we0_ref[...], preferred_element_type=jnp.float32)
    hd_ref[...] = jnp.dot(hb, we2_ref[...], preferred_element_type=jnp.float32)


def _tc_node_pre(x, W_np, b_np, We0, We2):
    blk = 1000
    return pl.pallas_call(
        _tc_node_pre_body,
        grid=(N // blk,),
        in_specs=[
            pl.BlockSpec((blk, D), lambda i: (i, 0)),
            pl.BlockSpec((D, DH), lambda i: (0, 0)),
            pl.BlockSpec((DH,), lambda i: (0,)),
            pl.BlockSpec((DH, DE), lambda i: (0, 0)),
            pl.BlockSpec((DH, DE), lambda i: (0, 0)),
        ],
        out_specs=[
            pl.BlockSpec((blk, DH), lambda i: (i, 0)),
            pl.BlockSpec((blk, DE), lambda i: (i, 0)),
            pl.BlockSpec((blk, DE), lambda i: (i, 0)),
        ],
        out_shape=[
            jax.ShapeDtypeStruct((N, DH), jnp.float32),
            jax.ShapeDtypeStruct((N, DE), jnp.float32),
            jax.ShapeDtypeStruct((N, DE), jnp.float32),
        ],
    )(x, W_np, b_np, We0, We2)


def _tc_edge_pre_body(ea_ref, wepb_ref, bep_ref, we1b_ref, bet_ref,
                      e_ref, ee_ref):
    eb = jnp.dot(ea_ref[...], wepb_ref[...],
                 preferred_element_type=jnp.float32) + bep_ref[...]
    e_ref[...] = eb
    ee_ref[...] = jnp.dot(eb, we1b_ref[...],
                          preferred_element_type=jnp.float32) + bet_ref[...]


def _tc_edge_pre(ea128, Wepb, bep8, We1b, bet8):
    eblk = 4000
    return pl.pallas_call(
        _tc_edge_pre_body,
        grid=(E // 8 // eblk,),
        in_specs=[
            pl.BlockSpec((eblk, D), lambda i: (i, 0)),
            pl.BlockSpec((D, D), lambda i: (0, 0)),
            pl.BlockSpec((D,), lambda i: (0,)),
            pl.BlockSpec((D, D), lambda i: (0, 0)),
            pl.BlockSpec((D,), lambda i: (0,)),
        ],
        out_specs=[
            pl.BlockSpec((eblk, D), lambda i: (i, 0)),
            pl.BlockSpec((eblk, D), lambda i: (i, 0)),
        ],
        out_shape=[
            jax.ShapeDtypeStruct((E // 8, D), jnp.float32),
            jax.ShapeDtypeStruct((E // 8, D), jnp.float32),
        ],
    )(ea128, Wepb, bep8, We1b, bet8)


_ZERO16 = functools.partial(jnp.zeros, (LANES,), jnp.float32)

NROWS = E // BLK
ROW_Q, ROW_R = NROWS // NS, NROWS % NS
EROW_Q, EROW_R = NROWS // (2 * NS), NROWS % (2 * NS)


def _sc_graph_body(h_hbm, e_hbm, src_hbm, dst_hbm, hs_hbm, hd_hbm, ee_hbm,
                   shi_hbm, sho_hbm, sei_hbm, seo_hbm, eo_hbm, deg_sh,
                   acc_h, acc_e,
                   rows_v, erow_v, epack_v, opack_v, gi8_v, si8_v,
                   hist_v, degr_v, tmp_v, b_v, sem, sem2):
    c = lax.axis_index("c")
    s = lax.axis_index("s")

    def zero_rows(i, _):
        for k in range(DH // LANES):
            rows_v[i, pl.ds(k * LANES, LANES)] = _ZERO16()
        erow_v[i] = _ZERO16()
        return ()

    lax.fori_loop(0, BLK, zero_rows, ())

    def zero_acc(g, _):
        r = s * ROWS_PT + g * BLK
        pltpu.sync_copy(rows_v, acc_h.at[pl.ds(r, BLK)])
        pltpu.sync_copy(erow_v, acc_e.at[pl.ds(r, BLK)])
        return ()

    lax.fori_loop(0, ROWS_PT // BLK, zero_acc, ())

    def zero_hist(i, _):
        hist_v[pl.ds(i * LANES, LANES)] = _ZERO16()
        return ()

    lax.fori_loop(0, N_PAD // LANES, zero_hist, ())
    plsc.subcore_barrier()

    my_start = s * ROW_Q + jnp.minimum(s, ROW_R)
    my_cnt = ROW_Q + jnp.where(s < ROW_R, 1, 0)

    def run_direction(gref, sref):
        def grp_body(g8, _):
            ld = jnp.minimum(my_start + g8 * 8, NROWS - 8)
            pltpu.sync_copy(gref.at[pl.ds(ld, 8)], gi8_v)
            pltpu.sync_copy(sref.at[pl.ds(ld, 8)], si8_v)

            def blk_body(j, _):
                bidx = g8 * 8 + j

                @pl.when(bidx < my_cnt)
                def _():
                    r = my_start + bidx
                    jj = r - ld
                    cp = pltpu.async_copy(h_hbm.at[gi8_v.at[jj]],
                                          rows_v, sem)
                    pltpu.sync_copy(e_hbm.at[pl.ds(r * (BLK // 8), BLK // 8)],
                                    epack_v)

                    def unpack_i(i, _):
                        for u in range(8):
                            erow_v[i * 8 + u] = epack_v[i, pl.ds(u * LANES,
                                                                 LANES)]
                        return ()

                    lax.fori_loop(0, BLK // 8, unpack_i, ())

                    for j2 in range(BLK // LANES):
                        iv = si8_v[jj, pl.ds(j2 * LANES, LANES)]
                        plsc.addupdate_scatter(
                            hist_v, [iv], jnp.ones((LANES,), jnp.float32))
                    cp.wait()
                    pltpu.sync_copy(rows_v, acc_h.at[si8_v.at[jj]], add=True)
                    pltpu.sync_copy(erow_v, acc_e.at[si8_v.at[jj]], add=True)

                return ()

            lax.fori_loop(0, 8, blk_body, ())
            return ()

        lax.fori_loop(0, (ROW_Q + 8) // 8, grp_body, ())

    @pl.when(c == 0)
    def _():
        run_direction(src_hbm, dst_hbm)

    @pl.when(c == 1)
    def _():
        run_direction(dst_hbm, src_hbm)

    pltpu.sync_copy(hist_v, deg_sh.at[c, s])
    plsc.subcore_barrier()

    r0 = s * ROWS_PT

    def red_init(i, _):
        degr_v[pl.ds(i * LANES, LANES)] = _ZERO16()
        return ()

    lax.fori_loop(0, ROWS_PT // LANES, red_init, ())

    def red_j(j, _):
        pltpu.sync_copy(deg_sh.at[c, j, pl.ds(r0, ROWS_PT)], tmp_v)

        def addk(k, _):
            sl = pl.ds(k * LANES, LANES)
            degr_v[sl] = degr_v[sl] + tmp_v[sl]
            return ()

        lax.fori_loop(0, ROWS_PT // LANES, addk, ())
        return ()

    lax.fori_loop(0, NS, red_j, ())

    def scale_big(g5, _):
        rr = r0 + g5 * BLK
        pltpu.sync_copy(acc_h.at[pl.ds(rr, BLK)], rows_v)
        pltpu.sync_copy(acc_e.at[pl.ds(rr, BLK)], erow_v)

        def scale_grp(gg, _):
            deg16 = degr_v[pl.ds(g5 * BLK + gg * LANES, LANES)]
            r16 = 1.0 / jnp.maximum(deg16, 1.0)
            for i in range(LANES):
                r_s = r16[i]
                row = gg * LANES + i
                for k in range(DH // LANES):
                    sl = pl.ds(k * LANES, LANES)
                    rows_v[row, sl] = rows_v[row, sl] * r_s
                erow_v[row] = erow_v[row] * r_s
            return ()

        lax.fori_loop(0, BLK // LANES, scale_grp, ())

        @pl.when(c == 0)
        def _():
            pltpu.sync_copy(rows_v, shi_hbm.at[pl.ds(rr, BLK)])
            pltpu.sync_copy(erow_v, sei_hbm.at[pl.ds(rr, BLK)])

        @pl.when(c == 1)
        def _():
            pltpu.sync_copy(rows_v, sho_hbm.at[pl.ds(rr, BLK)])
            pltpu.sync_copy(erow_v, seo_hbm.at[pl.ds(rr, BLK)])

        return ()

    lax.fori_loop(0, ROWS_PT // BLK, scale_big, ())

    wid = s * NC + c
    eo_start = wid * EROW_Q + jnp.minimum(wid, EROW_R)
    eo_cnt = EROW_Q + jnp.where(wid < EROW_R, 1, 0)

    def eo_grp(g8, _):
        ld = jnp.minimum(eo_start + g8 * 8, NROWS - 8)
        pltpu.sync_copy(src_hbm.at[pl.ds(ld, 8)], gi8_v)
        pltpu.sync_copy(dst_hbm.at[pl.ds(ld, 8)], si8_v)

        def eo_body(j, _):
            bidx = g8 * 8 + j

            @pl.when(bidx < eo_cnt)
            def _():
                r = eo_start + bidx
                jj = r - ld
                cp1 = pltpu.async_copy(hs_hbm.at[gi8_v.at[jj]], erow_v, sem)
                cp2 = pltpu.async_copy(hd_hbm.at[si8_v.at[jj]], b_v, sem2)
                pltpu.sync_copy(ee_hbm.at[pl.ds(r * (BLK // 8), BLK // 8)],
                                epack_v)
                cp1.wait()
                cp2.wait()

                def packadd_i(i, _):
                    for u in range(8):
                        sl = pl.ds(u * LANES, LANES)
                        opack_v[i, sl] = (epack_v[i, sl]
                                          + erow_v[i * 8 + u] + b_v[i * 8 + u])
                    return ()

                lax.fori_loop(0, BLK // 8, packadd_i, ())
                pltpu.sync_copy(opack_v,
                                eo_hbm.at[pl.ds(r * (BLK // 8), BLK // 8)])

            return ()

        lax.fori_loop(0, 8, eo_body, ())
        return ()

    lax.fori_loop(0, (EROW_Q + 8) // 8, eo_grp, ())


_sc_graph = functools.partial(
    pl.kernel,
    out_type=[
        jax.ShapeDtypeStruct((N_PAD, DH), jnp.float32),
        jax.ShapeDtypeStruct((N_PAD, DH), jnp.float32),
        jax.ShapeDtypeStruct((N_PAD, DEH), jnp.float32),
        jax.ShapeDtypeStruct((N_PAD, DEH), jnp.float32),
        jax.ShapeDtypeStruct((E // 8, D), jnp.float32),
        jax.ShapeDtypeStruct((NC, NS, N_PAD), jnp.float32),
    ],
    mesh=plsc.VectorSubcoreMesh(core_axis_name="c", subcore_axis_name="s"),
    compiler_params=pltpu.CompilerParams(
        needs_layout_passes=False, use_tc_tiling_on_sc=False),
    scratch_types=[
        pltpu.VMEM_SHARED((N_PAD, DH), jnp.float32),
        pltpu.VMEM_SHARED((N_PAD, DEH), jnp.float32),
        pltpu.VMEM((BLK, DH), jnp.float32),
        pltpu.VMEM((BLK, DEH), jnp.float32),
        pltpu.VMEM((BLK // 8, D), jnp.float32),
        pltpu.VMEM((BLK // 8, D), jnp.float32),
        pltpu.VMEM((8, BLK), jnp.int32),
        pltpu.VMEM((8, BLK), jnp.int32),
        pltpu.VMEM((N_PAD,), jnp.float32),
        pltpu.VMEM((ROWS_PT,), jnp.float32),
        pltpu.VMEM((ROWS_PT,), jnp.float32),
        pltpu.VMEM((BLK, DE), jnp.float32),
        pltpu.SemaphoreType.DMA,
        pltpu.SemaphoreType.DMA,
    ],
)(_sc_graph_body)


def _tc_node_post_body(shi_ref, sei_ref, h_ref, sho_ref, seo_ref,
                       w1_ref, w2_ref, w3_ref, w4_ref, w5_ref, bnt_ref,
                       out_ref):
    acc = jnp.dot(shi_ref[...], w1_ref[...], preferred_element_type=jnp.float32)
    acc = acc + jnp.dot(sei_ref[...], w2_ref[...],
                        preferred_element_type=jnp.float32)
    acc = acc + jnp.dot(h_ref[...], w3_ref[...],
                        preferred_element_type=jnp.float32)
    acc = acc + jnp.dot(sho_ref[...], w4_ref[...],
                        preferred_element_type=jnp.float32)
    acc = acc + jnp.dot(seo_ref[...], w5_ref[...],
                        preferred_element_type=jnp.float32)
    out_ref[...] = acc + bnt_ref[...]


def _tc_node_post(shi, sei, h, sho, seo, W1, W2, W3, W4, W5, b_nt):
    blk = 1000
    return pl.pallas_call(
        _tc_node_post_body,
        grid=(N // blk,),
        in_specs=[
            pl.BlockSpec((blk, DH), lambda i: (i, 0)),
            pl.BlockSpec((blk, DEH), lambda i: (i, 0)),
            pl.BlockSpec((blk, DH), lambda i: (i, 0)),
            pl.BlockSpec((blk, DH), lambda i: (i, 0)),
            pl.BlockSpec((blk, DEH), lambda i: (i, 0)),
            pl.BlockSpec((DH, D), lambda i: (0, 0)),
            pl.BlockSpec((DEH, D), lambda i: (0, 0)),
            pl.BlockSpec((DH, D), lambda i: (0, 0)),
            pl.BlockSpec((DH, D), lambda i: (0, 0)),
            pl.BlockSpec((DEH, D), lambda i: (0, 0)),
            pl.BlockSpec((D,), lambda i: (0,)),
        ],
        out_specs=pl.BlockSpec((blk, D), lambda i: (i, 0)),
        out_shape=jax.ShapeDtypeStruct((N, D), jnp.float32),
    )(shi, sei, h, sho, seo, W1, W2, W3, W4, W5, b_nt)


def kernel(x, edge_index, edge_attr, W_np, b_np, W_ep, b_ep,
           W_nt, b_nt, W_et, b_et):
    src2d = edge_index[0].reshape(NROWS, BLK)
    dst2d = edge_index[1].reshape(NROWS, BLK)
    ea128 = edge_attr.reshape(E // 8, D)

    eye8 = jnp.eye(8, dtype=jnp.float32)
    Wepb = jnp.kron(eye8, W_ep)
    We1b = jnp.kron(eye8, W_et[DH:DH + DEH])
    bep8 = jnp.tile(b_ep, 8)
    bet8 = jnp.tile(b_et, 8)

    h, hs_et, hd_et = _tc_node_pre(
        x, W_np, b_np, W_et[0:DH], W_et[DH + DEH:])
    e128, ee128 = _tc_edge_pre(ea128, Wepb, bep8, We1b, bet8)

    shi, sho, sei, seo, eo128, _ = _sc_graph(
        h, e128, src2d, dst2d, hs_et, hd_et, ee128)

    edge_out = eo128.reshape(E, DE)
    node_out = _tc_node_post(
        shi, sei, h, sho, seo,
        W_nt[0:DH], W_nt[DH:DH + DEH], W_nt[DH + DEH:2 * DH + DEH],
        W_nt[2 * DH + DEH:3 * DH + DEH], W_nt[3 * DH + DEH:], b_nt)

    return node_out, edge_out


# speedup vs baseline: 1.7125x; 1.0013x over previous
import functools

import jax
import jax.numpy as jnp
from jax import lax
from jax.experimental import pallas as pl
from jax.experimental.pallas import tpu as pltpu
from jax.experimental.pallas import tpu_sc as plsc

N = 10000
E = 320000
D = 128
DE = 16
DH = 128
DEH = 16

NC = 2
NS = 16
LANES = 16

N_PAD = 10240
BLK = 128
ROWS_PT = N_PAD // NS


def _tc_node_pre_body(x_ref, wnp_ref, bnp_ref, we0_ref, we2_ref,
                      h_ref, hs_ref, hd_ref):
    hb = jnp.dot(x_ref[...], wnp_ref[...],
                 preferred_element_type=jnp.float32) + bnp_ref[...]
    h_ref[...] = hb
    hs_ref[...] = jnp.dot(hb, we0_ref[...], preferred_element_type=jnp.float32)
    hd_ref[...] = jnp.dot(hb, we2_ref[...], preferred_element_type=jnp.float32)


def _tc_node_pre(x, W_np, b_np, We0, We2):
    blk = 1000
    return pl.pallas_call(
        _tc_node_pre_body,
        grid=(N // blk,),
        in_specs=[
            pl.BlockSpec((blk, D), lambda i: (i, 0)),
            pl.BlockSpec((D, DH), lambda i: (0, 0)),
            pl.BlockSpec((DH,), lambda i: (0,)),
            pl.BlockSpec((DH, DE), lambda i: (0, 0)),
            pl.BlockSpec((DH, DE), lambda i: (0, 0)),
        ],
        out_specs=[
            pl.BlockSpec((blk, DH), lambda i: (i, 0)),
            pl.BlockSpec((blk, DE), lambda i: (i, 0)),
            pl.BlockSpec((blk, DE), lambda i: (i, 0)),
        ],
        out_shape=[
            jax.ShapeDtypeStruct((N, DH), jnp.float32),
            jax.ShapeDtypeStruct((N, DE), jnp.float32),
            jax.ShapeDtypeStruct((N, DE), jnp.float32),
        ],
    )(x, W_np, b_np, We0, We2)


def _tc_edge_pre_body(ea_ref, wepb_ref, bep_ref, we1b_ref, bet_ref,
                      e_ref, ee_ref):
    eb = jnp.dot(ea_ref[...], wepb_ref[...],
                 preferred_element_type=jnp.float32) + bep_ref[...]
    e_ref[...] = eb
    ee_ref[...] = jnp.dot(eb, we1b_ref[...],
                          preferred_element_type=jnp.float32) + bet_ref[...]


def _tc_edge_pre(ea128, Wepb, bep8, We1b, bet8):
    eblk = 4000
    return pl.pallas_call(
        _tc_edge_pre_body,
        grid=(E // 8 // eblk,),
        in_specs=[
            pl.BlockSpec((eblk, D), lambda i: (i, 0)),
            pl.BlockSpec((D, D), lambda i: (0, 0)),
            pl.BlockSpec((D,), lambda i: (0,)),
            pl.BlockSpec((D, D), lambda i: (0, 0)),
            pl.BlockSpec((D,), lambda i: (0,)),
        ],
        out_specs=[
            pl.BlockSpec((eblk, D), lambda i: (i, 0)),
            pl.BlockSpec((eblk, D), lambda i: (i, 0)),
        ],
        out_shape=[
            jax.ShapeDtypeStruct((E // 8, D), jnp.float32),
            jax.ShapeDtypeStruct((E // 8, D), jnp.float32),
        ],
    )(ea128, Wepb, bep8, We1b, bet8)


_ZERO16 = functools.partial(jnp.zeros, (LANES,), jnp.float32)

NROWS = E // BLK
ROW_Q, ROW_R = NROWS // NS, NROWS % NS
EROW_Q, EROW_R = NROWS // (2 * NS), NROWS % (2 * NS)


def _sc_graph_body(h_hbm, e_hbm, src_hbm, dst_hbm, hs_hbm, hd_hbm, ee_hbm,
                   shi_hbm, sho_hbm, sei_hbm, seo_hbm, eo_hbm, deg_sh,
                   acc_h, acc_e,
                   rows_v, erow_v, epack_v, opack_v, gi8_v, si8_v,
                   hist_v, degr_v, tmp_v, b_v, sem, sem2):
    c = lax.axis_index("c")
    s = lax.axis_index("s")

    def zero_rows(i, _):
        for k in range(DH // LANES):
            rows_v[i, pl.ds(k * LANES, LANES)] = _ZERO16()
        erow_v[i] = _ZERO16()
        return ()

    lax.fori_loop(0, BLK, zero_rows, ())

    def zero_acc(g, _):
        r = s * ROWS_PT + g * BLK
        pltpu.sync_copy(rows_v, acc_h.at[pl.ds(r, BLK)])
        pltpu.sync_copy(erow_v, acc_e.at[pl.ds(r, BLK)])
        return ()

    lax.fori_loop(0, ROWS_PT // BLK, zero_acc, ())

    def zero_hist(i, _):
        hist_v[pl.ds(i * LANES, LANES)] = _ZERO16()
        return ()

    lax.fori_loop(0, N_PAD // LANES, zero_hist, ())
    plsc.subcore_barrier()

    my_start = s * ROW_Q + jnp.minimum(s, ROW_R)
    my_cnt = ROW_Q + jnp.where(s < ROW_R, 1, 0)

    def run_direction(gref, sref):
        def grp_body(g8, _):
            ld = jnp.minimum(my_start + g8 * 8, NROWS - 8)
            pltpu.sync_copy(gref.at[pl.ds(ld, 8)], gi8_v)
            pltpu.sync_copy(sref.at[pl.ds(ld, 8)], si8_v)

            def blk_body(j, _):
                bidx = g8 * 8 + j

                @pl.when(bidx < my_cnt)
                def _():
                    r = my_start + bidx
                    jj = r - ld
                    cp = pltpu.async_copy(h_hbm.at[gi8_v.at[jj]],
                                          rows_v, sem)
                    pltpu.sync_copy(e_hbm.at[pl.ds(r * (BLK // 8), BLK // 8)],
                                    epack_v)

                    def unpack_i(i, _):
                        for u in range(8):
                            erow_v[i * 8 + u] = epack_v[i, pl.ds(u * LANES,
                                                                 LANES)]
                        return ()

                    lax.fori_loop(0, BLK // 8, unpack_i, ())

                    for j2 in range(BLK // LANES):
                        iv = si8_v[jj, pl.ds(j2 * LANES, LANES)]
                        plsc.addupdate_scatter(
                            hist_v, [iv], jnp.ones((LANES,), jnp.float32))
                    cp.wait()
                    pltpu.sync_copy(rows_v, acc_h.at[si8_v.at[jj]], add=True)
                    pltpu.sync_copy(erow_v, acc_e.at[si8_v.at[jj]], add=True)

                return ()

            lax.fori_loop(0, 8, blk_body, ())
            return ()

        lax.fori_loop(0, (ROW_Q + 8) // 8, grp_body, ())

    @pl.when(c == 0)
    def _():
        run_direction(src_hbm, dst_hbm)

    @pl.when(c == 1)
    def _():
        run_direction(dst_hbm, src_hbm)

    pltpu.sync_copy(hist_v, deg_sh.at[c, s])
    plsc.subcore_barrier()

    r0 = s * ROWS_PT

    def red_init(i, _):
        degr_v[pl.ds(i * LANES, LANES)] = _ZERO16()
        return ()

    lax.fori_loop(0, ROWS_PT // LANES, red_init, ())

    def red_j(j, _):
        pltpu.sync_copy(deg_sh.at[c, j, pl.ds(r0, ROWS_PT)], tmp_v)

        def addk(k, _):
            sl = pl.ds(k * LANES, LANES)
            degr_v[sl] = degr_v[sl] + tmp_v[sl]
            return ()

        lax.fori_loop(0, ROWS_PT // LANES, addk, ())
        return ()

    lax.fori_loop(0, NS, red_j, ())

    def scale_big(g5, _):
        rr = r0 + g5 * BLK
        pltpu.sync_copy(acc_h.at[pl.ds(rr, BLK)], rows_v)
        pltpu.sync_copy(acc_e.at[pl.ds(rr, BLK)], erow_v)

        def scale_grp(gg, _):
            deg16 = degr_v[pl.ds(g5 * BLK + gg * LANES, LANES)]
            r16 = 1.0 / jnp.maximum(deg16, 1.0)
            for i in range(LANES):
                r_s = r16[i]
                row = gg * LANES + i
                for k in range(DH // LANES):
                    sl = pl.ds(k * LANES, LANES)
                    rows_v[row, sl] = rows_v[row, sl] * r_s
                erow_v[row] = erow_v[row] * r_s
            return ()

        lax.fori_loop(0, BLK // LANES, scale_grp, ())

        @pl.when(c == 0)
        def _():
            pltpu.sync_copy(rows_v, shi_hbm.at[pl.ds(rr, BLK)])
            pltpu.sync_copy(erow_v, sei_hbm.at[pl.ds(rr, BLK)])

        @pl.when(c == 1)
        def _():
            pltpu.sync_copy(rows_v, sho_hbm.at[pl.ds(rr, BLK)])
            pltpu.sync_copy(erow_v, seo_hbm.at[pl.ds(rr, BLK)])

        return ()

    lax.fori_loop(0, ROWS_PT // BLK, scale_big, ())

    wid = s * NC + c
    eo_start = wid * EROW_Q + jnp.minimum(wid, EROW_R)
    eo_cnt = EROW_Q + jnp.where(wid < EROW_R, 1, 0)

    def eo_grp(g8, _):
        ld = jnp.minimum(eo_start + g8 * 8, NROWS - 8)
        pltpu.sync_copy(src_hbm.at[pl.ds(ld, 8)], gi8_v)
        pltpu.sync_copy(dst_hbm.at[pl.ds(ld, 8)], si8_v)

        def eo_body(j, _):
            bidx = g8 * 8 + j

            @pl.when(bidx < eo_cnt)
            def _():
                r = eo_start + bidx
                jj = r - ld
                cp1 = pltpu.async_copy(hs_hbm.at[gi8_v.at[jj]], erow_v, sem)
                cp2 = pltpu.async_copy(hd_hbm.at[si8_v.at[jj]], b_v, sem2)
                pltpu.sync_copy(ee_hbm.at[pl.ds(r * (BLK // 8), BLK // 8)],
                                epack_v)
                cp1.wait()
                cp2.wait()

                def packadd_i(i, _):
                    for u in range(8):
                        sl = pl.ds(u * LANES, LANES)
                        opack_v[i, sl] = (epack_v[i, sl]
                                          + erow_v[i * 8 + u] + b_v[i * 8 + u])
                    return ()

                lax.fori_loop(0, BLK // 8, packadd_i, ())
                pltpu.sync_copy(opack_v,
                                eo_hbm.at[pl.ds(r * (BLK // 8), BLK // 8)])

            return ()

        lax.fori_loop(0, 8, eo_body, ())
        return ()

    lax.fori_loop(0, (EROW_Q + 8) // 8, eo_grp, ())


_sc_graph = functools.partial(
    pl.kernel,
    out_type=[
        jax.ShapeDtypeStruct((N_PAD, DH), jnp.float32),
        jax.ShapeDtypeStruct((N_PAD, DH), jnp.float32),
        jax.ShapeDtypeStruct((N_PAD, DEH), jnp.float32),
        jax.ShapeDtypeStruct((N_PAD, DEH), jnp.float32),
        jax.ShapeDtypeStruct((E // 8, D), jnp.float32),
        jax.ShapeDtypeStruct((NC, NS, N_PAD), jnp.float32),
    ],
    mesh=plsc.VectorSubcoreMesh(core_axis_name="c", subcore_axis_name="s"),
    compiler_params=pltpu.CompilerParams(
        needs_layout_passes=False, use_tc_tiling_on_sc=False),
    scratch_types=[
        pltpu.VMEM_SHARED((N_PAD, DH), jnp.float32),
        pltpu.VMEM_SHARED((N_PAD, DEH), jnp.float32),
        pltpu.VMEM((BLK, DH), jnp.float32),
        pltpu.VMEM((BLK, DEH), jnp.float32),
        pltpu.VMEM((BLK // 8, D), jnp.float32),
        pltpu.VMEM((BLK // 8, D), jnp.float32),
        pltpu.VMEM((8, BLK), jnp.int32),
        pltpu.VMEM((8, BLK), jnp.int32),
        pltpu.VMEM((N_PAD,), jnp.float32),
        pltpu.VMEM((ROWS_PT,), jnp.float32),
        pltpu.VMEM((ROWS_PT,), jnp.float32),
        pltpu.VMEM((BLK, DE), jnp.float32),
        pltpu.SemaphoreType.DMA,
        pltpu.SemaphoreType.DMA,
    ],
)(_sc_graph_body)


def _tc_node_post_body(shi_ref, sei_ref, h_ref, sho_ref, seo_ref,
                       w1_ref, w2_ref, w3_ref, w4_ref, w5_ref, bnt_ref,
                       out_ref):
    acc = jnp.dot(shi_ref[...], w1_ref[...], preferred_element_type=jnp.float32)
    acc = acc + jnp.dot(sei_ref[...], w2_ref[...],
                        preferred_element_type=jnp.float32)
    acc = acc + jnp.dot(h_ref[...], w3_ref[...],
                        preferred_element_type=jnp.float32)
    acc = acc + jnp.dot(sho_ref[...], w4_ref[...],
                        preferred_element_type=jnp.float32)
    acc = acc + jnp.dot(seo_ref[...], w5_ref[...],
                        preferred_element_type=jnp.float32)
    out_ref[...] = acc + bnt_ref[...]


def _tc_node_post(shi, sei, h, sho, seo, W1, W2, W3, W4, W5, b_nt):
    blk = 1000
    return pl.pallas_call(
        _tc_node_post_body,
        grid=(N // blk,),
        in_specs=[
            pl.BlockSpec((blk, DH), lambda i: (i, 0)),
            pl.BlockSpec((blk, DEH), lambda i: (i, 0)),
            pl.BlockSpec((blk, DH), lambda i: (i, 0)),
            pl.BlockSpec((blk, DH), lambda i: (i, 0)),
            pl.BlockSpec((blk, DEH), lambda i: (i, 0)),
            pl.BlockSpec((DH, D), lambda i: (0, 0)),
            pl.BlockSpec((DEH, D), lambda i: (0, 0)),
            pl.BlockSpec((DH, D), lambda i: (0, 0)),
            pl.BlockSpec((DH, D), lambda i: (0, 0)),
            pl.BlockSpec((DEH, D), lambda i: (0, 0)),
            pl.BlockSpec((D,), lambda i: (0,)),
        ],
        out_specs=pl.BlockSpec((blk, D), lambda i: (i, 0)),
        out_shape=jax.ShapeDtypeStruct((N, D), jnp.float32),
    )(shi, sei, h, sho, seo, W1, W2, W3, W4, W5, b_nt)


def kernel(x, edge_index, edge_attr, W_np, b_np, W_ep, b_ep,
           W_nt, b_nt, W_et, b_et):
    src2d = edge_index[0].reshape(NROWS, BLK)
    dst2d = edge_index[1].reshape(NROWS, BLK)
    ea128 = edge_attr.reshape(E // 8, D)

    eye8 = jnp.eye(8, dtype=jnp.float32)
    Wepb = jnp.kron(eye8, W_ep)
    We1b = jnp.kron(eye8, W_et[DH:DH + DEH])
    bep8 = jnp.tile(b_ep, 8)
    bet8 = jnp.tile(b_et, 8)

    h, hs_et, hd_et = _tc_node_pre(
        x, W_np, b_np, W_et[0:DH], W_et[DH + DEH:])
    e128, ee128 = _tc_edge_pre(ea128, Wepb, bep8, We1b, bet8)

    shi, sho, sei, seo, eo128, _ = _sc_graph(
        h, e128, src2d, dst2d, hs_et, hd_et, ee128)

    edge_out = eo128.reshape(E, DE)
    node_out = _tc_node_post(
        shi, sei, h, sho, seo,
        W_nt[0:DH], W_nt[DH:DH + DEH], W_nt[DH + DEH:2 * DH + DEH],
        W_nt[2 * DH + DEH:3 * DH + DEH], W_nt[3 * DH + DEH:], b_nt)

    return node_out, edge_out
